# Initial kernel scaffold; baseline (speedup 1.0000x reference)
import jax
import jax.numpy as jnp
from jax.experimental import pallas as pl


def kernel(edge_pos, edge_neg, edge_type, node_emb, conv1_weight, conv1_root, conv1_bias, conv2_weight, conv2_root, conv2_bias, rel_emb):
    raise NotImplementedError("write your pallas kernel here")


# speedup vs baseline: 1.2709x; 1.2709x over previous
import functools

import jax
import jax.numpy as jnp
from jax import lax
from jax.experimental import pallas as pl
from jax.experimental.pallas import tpu as pltpu
from jax.experimental.pallas import tpu_sc as plsc

N_ENT = 10000
N_REL = 16
D = 128
N_BLOCKS = 4
E = 320000

NC = 2
NS = 16
NW = NC * NS

_MESH = plsc.VectorSubcoreMesh(core_axis_name="c", subcore_axis_name="s")


def _worker_id():
    return lax.axis_index("s") * NC + lax.axis_index("c")


def _gather4(table, i0, i1, i2, i3):
    C = 80
    per_w = E // NW
    n_chunks = per_w // C

    out_sd = jax.ShapeDtypeStruct((E, D), jnp.float32)

    @functools.partial(
        pl.kernel,
        out_type=(out_sd,) * 4,
        mesh=_MESH,
        scratch_types=[
            pltpu.VMEM((C,), jnp.int32),
            pltpu.VMEM((C, D), jnp.float32),
            pltpu.SemaphoreType.DMA,
        ],
    )
    def k(tab_h, i0_h, i1_h, i2_h, i3_h, o0_h, o1_h, o2_h, o3_h,
          idx_v, rows_v, sem):
        base = _worker_id() * per_w
        for idx_h, out_h in ((i0_h, o0_h), (i1_h, o1_h),
                             (i2_h, o2_h), (i3_h, o3_h)):
            def body(i, _, idx_h=idx_h, out_h=out_h):
                off = base + i * C
                pltpu.sync_copy(idx_h.at[pl.ds(off, C)], idx_v)
                pltpu.async_copy(tab_h.at[idx_v], rows_v, sem).wait()
                pltpu.sync_copy(rows_v, out_h.at[pl.ds(off, C)])
                return 0
            lax.fori_loop(0, n_chunks, body, 0)

    return k(table, i0, i1, i2, i3)


def _rgcn_conv(x, edge_index, edge_type, weight, root, bias):
    src = edge_index[0]
    dst = edge_index[1]
    msgs = jnp.take(x, src, axis=0)
    seg = dst * N_REL + edge_type
    agg = jax.ops.segment_sum(msgs, seg, num_segments=N_ENT * N_REL)
    cnt = jax.ops.segment_sum(jnp.ones((src.shape[0],), x.dtype), seg,
                              num_segments=N_ENT * N_REL)
    mean = agg / jnp.maximum(cnt, 1.0)[:, None]
    bs = D // N_BLOCKS
    mean = mean.reshape(N_ENT, N_REL, N_BLOCKS, bs)
    out = jnp.einsum('nrbi,rbio->nbo', mean, weight).reshape(N_ENT, D)
    return out + x @ root + bias


def kernel(edge_pos, edge_neg, edge_type, node_emb, conv1_weight, conv1_root,
           conv1_bias, conv2_weight, conv2_root, conv2_bias, rel_emb):
    x = _rgcn_conv(node_emb, edge_pos, edge_type, conv1_weight, conv1_root,
                   conv1_bias)
    x = jax.nn.relu(x)
    z = _rgcn_conv(x, edge_pos, edge_type, conv2_weight, conv2_root,
                   conv2_bias)

    zs_p, zd_p, zs_n, zd_n = _gather4(
        z, edge_pos[0], edge_pos[1], edge_neg[0], edge_neg[1])
    rel = jnp.take(rel_emb, edge_type, axis=0)
    pos_out = jnp.sum(zs_p * rel * zd_p, axis=1)
    neg_out = jnp.sum(zs_n * rel * zd_n, axis=1)

    ce = 0.5 * (jnp.mean(jax.nn.softplus(-pos_out))
                + jnp.mean(jax.nn.softplus(neg_out)))
    reg = jnp.mean(z ** 2) + jnp.mean(rel_emb ** 2)
    return ce + 0.01 * reg


# speedup vs baseline: 2.8846x; 2.2698x over previous
import functools

import jax
import jax.numpy as jnp
from jax import lax
from jax.experimental import pallas as pl
from jax.experimental.pallas import tpu as pltpu
from jax.experimental.pallas import tpu_sc as plsc

N_ENT = 10000
N_REL = 16
D = 128
N_BLOCKS = 4
E = 320000

NC = 2
NS = 16
NW = NC * NS

_MESH = plsc.VectorSubcoreMesh(core_axis_name="c", subcore_axis_name="s")


def _worker_id():
    return lax.axis_index("s") * NC + lax.axis_index("c")


def _gather4(table, i0, i1, i2, i3):
    C = 80
    per_w = E // NW
    n_chunks = per_w // C

    out_sd = jax.ShapeDtypeStruct((E, D), jnp.float32)

    @functools.partial(
        pl.kernel,
        out_type=(out_sd,) * 4,
        mesh=_MESH,
        scratch_types=[
            pltpu.VMEM((C,), jnp.int32),
            pltpu.VMEM((C, D), jnp.float32),
            pltpu.SemaphoreType.DMA,
        ],
    )
    def k(tab_h, i0_h, i1_h, i2_h, i3_h, o0_h, o1_h, o2_h, o3_h,
          idx_v, rows_v, sem):
        base = _worker_id() * per_w
        for idx_h, out_h in ((i0_h, o0_h), (i1_h, o1_h),
                             (i2_h, o2_h), (i3_h, o3_h)):
            def body(i, _, idx_h=idx_h, out_h=out_h):
                off = base + i * C
                pltpu.sync_copy(idx_h.at[pl.ds(off, C)], idx_v)
                pltpu.async_copy(tab_h.at[idx_v], rows_v, sem).wait()
                pltpu.sync_copy(rows_v, out_h.at[pl.ds(off, C)])
                return 0
            lax.fori_loop(0, n_chunks, body, 0)

    return k(table, i0, i1, i2, i3)


NSEG = N_ENT * N_REL
SEG_W = NSEG // NS
CNT_C = 80
STAGE_C = 2000


def _seg_counts(seg, zeros_seg):
    per_w = E // NW
    n_chunks = per_w // CNT_C

    @functools.partial(
        pl.kernel,
        out_type=(jax.ShapeDtypeStruct((NSEG,), jnp.float32),) * 2,
        mesh=_MESH,
        scratch_types=[
            pltpu.VMEM((CNT_C,), jnp.int32),
            pltpu.VMEM((CNT_C,), jnp.float32),
            pltpu.VMEM((STAGE_C,), jnp.float32),
            pltpu.VMEM_SHARED((NSEG,), jnp.float32),
        ],
    )
    def k(seg_h, zero_h, cnt0_h, cnt1_h, seg_v, ones_v, stage_v, cnt_sh):
        cid = lax.axis_index("c")
        sid = lax.axis_index("s")
        base = _worker_id() * per_w

        for j in range(CNT_C // 16):
            ones_v[pl.ds(j * 16, 16)] = jnp.ones((16,), jnp.float32)

        def zstage(i, _):
            stage_v[pl.ds(i * 16, 16)] = jnp.zeros((16,), jnp.float32)
            return 0
        lax.fori_loop(0, STAGE_C // 16, zstage, 0)
        for j in range(SEG_W // STAGE_C):
            pltpu.sync_copy(
                stage_v, cnt_sh.at[pl.ds(sid * SEG_W + j * STAGE_C, STAGE_C)])
        plsc.subcore_barrier()

        def body(i, _):
            pltpu.sync_copy(seg_h.at[pl.ds(base + i * CNT_C, CNT_C)], seg_v)
            pltpu.sync_copy(ones_v, cnt_sh.at[seg_v], add=True)
            return 0
        lax.fori_loop(0, n_chunks, body, 0)

        plsc.subcore_barrier()
        for j in range(SEG_W // STAGE_C):
            off = sid * SEG_W + j * STAGE_C
            pltpu.sync_copy(cnt_sh.at[pl.ds(off, STAGE_C)], stage_v)

            @pl.when(cid == 0)
            def _(off=off):
                pltpu.sync_copy(stage_v, cnt0_h.at[pl.ds(off, STAGE_C)])

            @pl.when(cid == 1)
            def _(off=off):
                pltpu.sync_copy(stage_v, cnt1_h.at[pl.ds(off, STAGE_C)])

    return k(seg, zeros_seg)


AGG_C = 80
DUMP_C = 200
DUMP_N = N_ENT // DUMP_C


def _conv_agg(t_flat, gidx, seg, dst, inv, zeros_nd):
    per_w = E // NW
    n_chunks = per_w // AGG_C

    @functools.partial(
        pl.kernel,
        out_type=(jax.ShapeDtypeStruct((N_ENT, D), jnp.float32),) * 2,
        mesh=_MESH,
        scratch_types=[
            pltpu.VMEM((AGG_C,), jnp.int32),
            pltpu.VMEM((AGG_C,), jnp.int32),
            pltpu.VMEM((AGG_C,), jnp.int32),
            pltpu.VMEM((AGG_C,), jnp.float32),
            pltpu.VMEM((AGG_C, D), jnp.float32),
            pltpu.VMEM((DUMP_C, D), jnp.float32),
            pltpu.VMEM_SHARED((N_ENT, D), jnp.float32),
            pltpu.SemaphoreType.DMA,
        ],
    )
    def k(t_h, gidx_h, seg_h, dst_h, inv_h, zero_h, out0_h, out1_h,
          gidx_v, seg_v, dst_v, scale_v, rows_v, stage_v,
          acc_sh, sem):
        cid = lax.axis_index("c")
        sid = lax.axis_index("s")
        base = _worker_id() * per_w

        pltpu.sync_copy(zero_h.at[pl.ds(0, DUMP_C)], stage_v)
        for j in range(DUMP_N // NS + 1):
            c = sid + j * NS

            @pl.when(c < DUMP_N)
            def _(c=c):
                off = pl.multiple_of(c * DUMP_C, 8)
                pltpu.sync_copy(stage_v, acc_sh.at[pl.ds(off, DUMP_C)])
        plsc.subcore_barrier()

        def body(i, _):
            off = base + i * AGG_C
            pltpu.sync_copy(gidx_h.at[pl.ds(off, AGG_C)], gidx_v)
            pltpu.sync_copy(seg_h.at[pl.ds(off, AGG_C)], seg_v)
            pltpu.sync_copy(dst_h.at[pl.ds(off, AGG_C)], dst_v)
            pltpu.async_copy(t_h.at[gidx_v], rows_v, sem).wait()
            pltpu.async_copy(inv_h.at[seg_v], scale_v, sem).wait()

            def scale_grp(g, _):
                sv = scale_v[pl.ds(g * 16, 16)]

                def scale_body(l, _):
                    e = g * 16 + l
                    s = sv.at[jnp.full((16,), l, jnp.int32)].get(
                        mode="promise_in_bounds")
                    for j in range(D // 16):
                        rows_v[e, pl.ds(j * 16, 16)] = (
                            rows_v[e, pl.ds(j * 16, 16)] * s)
                    return 0
                lax.fori_loop(0, 16, scale_body, 0)
                return 0
            lax.fori_loop(0, AGG_C // 16, scale_grp, 0)

            pltpu.sync_copy(rows_v, acc_sh.at[dst_v], add=True)
            return 0
        lax.fori_loop(0, n_chunks, body, 0)

        plsc.subcore_barrier()
        for j in range(DUMP_N // NS + 1):
            c = sid + j * NS

            @pl.when(c < DUMP_N)
            def _(c=c):
                off = pl.multiple_of(c * DUMP_C, 8)
                pltpu.sync_copy(acc_sh.at[pl.ds(off, DUMP_C)], stage_v)

                @pl.when(cid == 0)
                def _():
                    pltpu.sync_copy(stage_v, out0_h.at[pl.ds(off, DUMP_C)])

                @pl.when(cid == 1)
                def _():
                    pltpu.sync_copy(stage_v, out1_h.at[pl.ds(off, DUMP_C)])

    return k(t_flat, gidx, seg, dst, inv, zeros_nd)


def _rgcn_conv_sc(x, gidx, seg, dst, inv, zeros_nd, weight, root, bias):
    bs = D // N_BLOCKS
    t = jnp.einsum('nbi,rbio->rnbo', x.reshape(N_ENT, N_BLOCKS, bs), weight)
    t_flat = t.reshape(N_REL * N_ENT, D)
    p0, p1 = _conv_agg(t_flat, gidx, seg, dst, inv, zeros_nd)
    return p0 + p1 + x @ root + bias


def kernel(edge_pos, edge_neg, edge_type, node_emb, conv1_weight, conv1_root,
           conv1_bias, conv2_weight, conv2_root, conv2_bias, rel_emb):
    src = edge_pos[0]
    dst = edge_pos[1]
    seg = dst * N_REL + edge_type
    gidx = edge_type * N_ENT + src
    zeros_nd = jnp.zeros((N_ENT, D), jnp.float32)

    c0, c1 = _seg_counts(seg, jnp.zeros((NSEG,), jnp.float32))
    inv = 1.0 / jnp.maximum(c0 + c1, 1.0)

    x = _rgcn_conv_sc(node_emb, gidx, seg, dst, inv, zeros_nd,
                      conv1_weight, conv1_root, conv1_bias)
    x = jax.nn.relu(x)
    z = _rgcn_conv_sc(x, gidx, seg, dst, inv, zeros_nd,
                      conv2_weight, conv2_root, conv2_bias)

    zs_p, zd_p, zs_n, zd_n = _gather4(
        z, edge_pos[0], edge_pos[1], edge_neg[0], edge_neg[1])
    rel = jnp.take(rel_emb, edge_type, axis=0)
    pos_out = jnp.sum(zs_p * rel * zd_p, axis=1)
    neg_out = jnp.sum(zs_n * rel * zd_n, axis=1)

    ce = 0.5 * (jnp.mean(jax.nn.softplus(-pos_out))
                + jnp.mean(jax.nn.softplus(neg_out)))
    reg = jnp.mean(z ** 2) + jnp.mean(rel_emb ** 2)
    return ce + 0.01 * reg


# speedup vs baseline: 3.2419x; 1.1239x over previous
import functools

import jax
import jax.numpy as jnp
from jax import lax
from jax.experimental import pallas as pl
from jax.experimental.pallas import tpu as pltpu
from jax.experimental.pallas import tpu_sc as plsc

N_ENT = 10000
N_REL = 16
D = 128
N_BLOCKS = 4
E = 320000

NC = 2
NS = 16
NW = NC * NS

_MESH = plsc.VectorSubcoreMesh(core_axis_name="c", subcore_axis_name="s")


def _worker_id():
    return lax.axis_index("s") * NC + lax.axis_index("c")


def _gather4(table, i0, i1, i2, i3):
    C = 80
    per_w = E // NW
    n_chunks = per_w // C

    out_sd = jax.ShapeDtypeStruct((E, D), jnp.float32)

    @functools.partial(
        pl.kernel,
        out_type=(out_sd,) * 4,
        mesh=_MESH,
        scratch_types=[
            pltpu.VMEM((C,), jnp.int32),
            pltpu.VMEM((C, D), jnp.float32),
            pltpu.SemaphoreType.DMA,
        ],
    )
    def k(tab_h, i0_h, i1_h, i2_h, i3_h, o0_h, o1_h, o2_h, o3_h,
          idx_v, rows_v, sem):
        base = _worker_id() * per_w
        for idx_h, out_h in ((i0_h, o0_h), (i1_h, o1_h),
                             (i2_h, o2_h), (i3_h, o3_h)):
            def body(i, _, idx_h=idx_h, out_h=out_h):
                off = base + i * C
                pltpu.sync_copy(idx_h.at[pl.ds(off, C)], idx_v)
                pltpu.async_copy(tab_h.at[idx_v], rows_v, sem).wait()
                pltpu.sync_copy(rows_v, out_h.at[pl.ds(off, C)])
                return 0
            lax.fori_loop(0, n_chunks, body, 0)

    return k(table, i0, i1, i2, i3)


NSEG = N_ENT * N_REL
SEG_W = NSEG // NS
CNT_C = 80
STAGE_C = 2000


def _seg_counts(seg, zeros_seg):
    per_w = E // NW
    n_chunks = per_w // CNT_C

    @functools.partial(
        pl.kernel,
        out_type=(jax.ShapeDtypeStruct((NSEG,), jnp.float32),) * 2,
        mesh=_MESH,
        scratch_types=[
            pltpu.VMEM((CNT_C,), jnp.int32),
            pltpu.VMEM((CNT_C,), jnp.float32),
            pltpu.VMEM((STAGE_C,), jnp.float32),
            pltpu.VMEM_SHARED((NSEG,), jnp.float32),
        ],
    )
    def k(seg_h, zero_h, cnt0_h, cnt1_h, seg_v, ones_v, stage_v, cnt_sh):
        cid = lax.axis_index("c")
        sid = lax.axis_index("s")
        base = _worker_id() * per_w

        for j in range(CNT_C // 16):
            ones_v[pl.ds(j * 16, 16)] = jnp.ones((16,), jnp.float32)

        def zstage(i, _):
            stage_v[pl.ds(i * 16, 16)] = jnp.zeros((16,), jnp.float32)
            return 0
        lax.fori_loop(0, STAGE_C // 16, zstage, 0)
        for j in range(SEG_W // STAGE_C):
            pltpu.sync_copy(
                stage_v, cnt_sh.at[pl.ds(sid * SEG_W + j * STAGE_C, STAGE_C)])
        plsc.subcore_barrier()

        def body(i, _):
            pltpu.sync_copy(seg_h.at[pl.ds(base + i * CNT_C, CNT_C)], seg_v)
            pltpu.sync_copy(ones_v, cnt_sh.at[seg_v], add=True)
            return 0
        lax.fori_loop(0, n_chunks, body, 0)

        plsc.subcore_barrier()
        for j in range(SEG_W // STAGE_C):
            off = sid * SEG_W + j * STAGE_C
            pltpu.sync_copy(cnt_sh.at[pl.ds(off, STAGE_C)], stage_v)

            @pl.when(cid == 0)
            def _(off=off):
                pltpu.sync_copy(stage_v, cnt0_h.at[pl.ds(off, STAGE_C)])

            @pl.when(cid == 1)
            def _(off=off):
                pltpu.sync_copy(stage_v, cnt1_h.at[pl.ds(off, STAGE_C)])

    return k(seg, zeros_seg)


AGG_C = 80
DUMP_C = 200
DUMP_N = N_ENT // DUMP_C


def _conv_agg(t_flat, gidx, seg, dst, inv, zeros_nd):
    per_w = E // NW
    n_chunks = per_w // AGG_C

    @functools.partial(
        pl.kernel,
        out_type=(jax.ShapeDtypeStruct((N_ENT, D), jnp.float32),) * 2,
        mesh=_MESH,
        scratch_types=[
            pltpu.VMEM((AGG_C,), jnp.int32),
            pltpu.VMEM((AGG_C,), jnp.int32),
            pltpu.VMEM((AGG_C,), jnp.int32),
            pltpu.VMEM((AGG_C,), jnp.float32),
            pltpu.VMEM((AGG_C, D), jnp.float32),
            pltpu.VMEM((DUMP_C, D), jnp.float32),
            pltpu.VMEM_SHARED((N_ENT, D), jnp.float32),
            pltpu.SemaphoreType.DMA,
        ],
    )
    def k(t_h, gidx_h, seg_h, dst_h, inv_h, zero_h, out0_h, out1_h,
          gidx_v, seg_v, dst_v, scale_v, rows_v, stage_v,
          acc_sh, sem):
        cid = lax.axis_index("c")
        sid = lax.axis_index("s")
        base = _worker_id() * per_w

        pltpu.sync_copy(zero_h.at[pl.ds(0, DUMP_C)], stage_v)
        for j in range(DUMP_N // NS + 1):
            c = sid + j * NS

            @pl.when(c < DUMP_N)
            def _(c=c):
                off = pl.multiple_of(c * DUMP_C, 8)
                pltpu.sync_copy(stage_v, acc_sh.at[pl.ds(off, DUMP_C)])
        plsc.subcore_barrier()

        def body(i, _):
            off = base + i * AGG_C
            pltpu.sync_copy(gidx_h.at[pl.ds(off, AGG_C)], gidx_v)
            pltpu.sync_copy(seg_h.at[pl.ds(off, AGG_C)], seg_v)
            pltpu.sync_copy(dst_h.at[pl.ds(off, AGG_C)], dst_v)
            pltpu.async_copy(t_h.at[gidx_v], rows_v, sem).wait()
            pltpu.async_copy(inv_h.at[seg_v], scale_v, sem).wait()

            def scale_grp(g, _):
                sv = scale_v[pl.ds(g * 16, 16)]

                def scale_body(l, _):
                    e = g * 16 + l
                    s = sv.at[jnp.full((16,), l, jnp.int32)].get(
                        mode="promise_in_bounds")
                    for j in range(D // 16):
                        rows_v[e, pl.ds(j * 16, 16)] = (
                            rows_v[e, pl.ds(j * 16, 16)] * s)
                    return 0
                lax.fori_loop(0, 16, scale_body, 0)
                return 0
            lax.fori_loop(0, AGG_C // 16, scale_grp, 0)

            pltpu.sync_copy(rows_v, acc_sh.at[dst_v], add=True)
            return 0
        lax.fori_loop(0, n_chunks, body, 0)

        plsc.subcore_barrier()
        for j in range(DUMP_N // NS + 1):
            c = sid + j * NS

            @pl.when(c < DUMP_N)
            def _(c=c):
                off = pl.multiple_of(c * DUMP_C, 8)
                pltpu.sync_copy(acc_sh.at[pl.ds(off, DUMP_C)], stage_v)

                @pl.when(cid == 0)
                def _():
                    pltpu.sync_copy(stage_v, out0_h.at[pl.ds(off, DUMP_C)])

                @pl.when(cid == 1)
                def _():
                    pltpu.sync_copy(stage_v, out1_h.at[pl.ds(off, DUMP_C)])

    return k(t_flat, gidx, seg, dst, inv, zeros_nd)


def _prep(x, wstack):
    def body(x_ref, w_ref, o_ref):
        o_ref[0] = jnp.dot(x_ref[...], w_ref[0],
                           preferred_element_type=jnp.float32)

    return pl.pallas_call(
        body,
        grid=(N_REL + 1,),
        in_specs=[
            pl.BlockSpec((N_ENT, D), lambda r: (0, 0)),
            pl.BlockSpec((1, D, D), lambda r: (r, 0, 0)),
        ],
        out_specs=pl.BlockSpec((1, N_ENT, D), lambda r: (r, 0, 0)),
        out_shape=jax.ShapeDtypeStruct((N_REL + 1, N_ENT, D), jnp.float32),
    )(x, wstack)


def _inv_counts(c0, c1):
    def body(a_ref, b_ref, o_ref):
        o_ref[...] = 1.0 / jnp.maximum(a_ref[...] + b_ref[...], 1.0)

    r = NSEG // D
    out = pl.pallas_call(
        body,
        out_shape=jax.ShapeDtypeStruct((r, D), jnp.float32),
    )(c0.reshape(r, D), c1.reshape(r, D))
    return out.reshape(NSEG)


def _combine(p0, p1, xr, bias, relu, with_ssq):
    def body(a_ref, b_ref, c_ref, bias_ref, o_ref, *maybe_ssq):
        v = a_ref[...] + b_ref[...] + c_ref[...] + bias_ref[...]
        if relu:
            v = jnp.maximum(v, 0.0)
        o_ref[...] = v
        if with_ssq:
            maybe_ssq[0][...] = jnp.sum(v * v).reshape(1, 1)

    out_shape = [jax.ShapeDtypeStruct((N_ENT, D), jnp.float32)]
    if with_ssq:
        out_shape.append(jax.ShapeDtypeStruct((1, 1), jnp.float32))
    res = pl.pallas_call(
        body,
        out_shape=tuple(out_shape),
    )(p0, p1, xr, bias.reshape(1, D))
    return res


LOSS_CH = 4000
LOSS_N = E // LOSS_CH


def _loss(zs_p, zd_p, zs_n, zd_n, edge_type, rel_emb):
    def body(sp_ref, dp_ref, sn_ref, dn_ref, et_ref, rel_ref, o_ref, r2_ref):
        i = pl.program_id(0)
        et = et_ref[0, 0, :]
        onehot = (et[:, None] ==
                  lax.broadcasted_iota(jnp.int32, (LOSS_CH, N_REL), 1)
                  ).astype(jnp.float32)
        rele = jnp.dot(onehot, rel_ref[...],
                       preferred_element_type=jnp.float32)
        s_pos = jnp.sum(sp_ref[...] * rele * dp_ref[...], axis=1)
        s_neg = jnp.sum(sn_ref[...] * rele * dn_ref[...], axis=1)
        part = (jnp.sum(jax.nn.softplus(-s_pos))
                + jnp.sum(jax.nn.softplus(s_neg)))

        @pl.when(i == 0)
        def _():
            o_ref[...] = jnp.zeros((1, 1), jnp.float32)
            r2_ref[...] = jnp.sum(rel_ref[...] * rel_ref[...]).reshape(1, 1)

        o_ref[...] += part.reshape(1, 1)

    row = pl.BlockSpec((LOSS_CH, D), lambda i: (i, 0))
    tot, relsq = pl.pallas_call(
        body,
        grid=(LOSS_N,),
        in_specs=[
            row, row, row, row,
            pl.BlockSpec((1, 1, LOSS_CH), lambda i: (i, 0, 0)),
            pl.BlockSpec((N_REL, D), lambda i: (0, 0)),
        ],
        out_specs=(pl.BlockSpec((1, 1), lambda i: (0, 0)),
                   pl.BlockSpec((1, 1), lambda i: (0, 0))),
        out_shape=(jax.ShapeDtypeStruct((1, 1), jnp.float32),
                   jax.ShapeDtypeStruct((1, 1), jnp.float32)),
    )(zs_p, zd_p, zs_n, zd_n, edge_type.reshape(LOSS_N, 1, LOSS_CH), rel_emb)
    return tot[0, 0], relsq[0, 0]


def _block_diag_stack(weight, root):
    bs = D // N_BLOCKS
    bd = jnp.zeros((N_REL, D, D), jnp.float32)
    for b in range(N_BLOCKS):
        bd = bd.at[:, b * bs:(b + 1) * bs, b * bs:(b + 1) * bs].set(
            weight[:, b])
    return jnp.concatenate([bd, root[None]], axis=0)


def _rgcn_conv_sc(x, gidx, seg, dst, inv, zeros_nd, weight, root, bias,
                  relu, with_ssq):
    s = _prep(x, _block_diag_stack(weight, root))
    t_flat = s[:N_REL].reshape(N_REL * N_ENT, D)
    p0, p1 = _conv_agg(t_flat, gidx, seg, dst, inv, zeros_nd)
    return _combine(p0, p1, s[N_REL], bias, relu, with_ssq)


def kernel(edge_pos, edge_neg, edge_type, node_emb, conv1_weight, conv1_root,
           conv1_bias, conv2_weight, conv2_root, conv2_bias, rel_emb):
    src = edge_pos[0]
    dst = edge_pos[1]
    seg = dst * N_REL + edge_type
    gidx = edge_type * N_ENT + src
    zeros_nd = jnp.zeros((N_ENT, D), jnp.float32)

    c0, c1 = _seg_counts(seg, jnp.zeros((NSEG,), jnp.float32))
    inv = _inv_counts(c0, c1)

    (x,) = _rgcn_conv_sc(node_emb, gidx, seg, dst, inv, zeros_nd,
                         conv1_weight, conv1_root, conv1_bias,
                         relu=True, with_ssq=False)
    z, ssq = _rgcn_conv_sc(x, gidx, seg, dst, inv, zeros_nd,
                           conv2_weight, conv2_root, conv2_bias,
                           relu=False, with_ssq=True)

    zs_p, zd_p, zs_n, zd_n = _gather4(
        z, edge_pos[0], edge_pos[1], edge_neg[0], edge_neg[1])
    tot, relsq = _loss(zs_p, zd_p, zs_n, zd_n, edge_type, rel_emb)

    ce = tot / (2.0 * E)
    reg = ssq[0, 0] / (N_ENT * D) + relsq / (N_REL * D)
    return ce + 0.01 * reg


# speedup vs baseline: 4.2758x; 1.3189x over previous
import functools

import jax
import jax.numpy as jnp
from jax import lax
from jax.experimental import pallas as pl
from jax.experimental.pallas import tpu as pltpu
from jax.experimental.pallas import tpu_sc as plsc

N_ENT = 10000
N_REL = 16
D = 128
N_BLOCKS = 4
E = 320000

NC = 2
NS = 16
NW = NC * NS

_MESH = plsc.VectorSubcoreMesh(core_axis_name="c", subcore_axis_name="s")


def _worker_id():
    return lax.axis_index("s") * NC + lax.axis_index("c")


def _gather4(table, i0, i1, i2, i3):
    C = 80
    per_w = E // NW
    n_chunks = per_w // C

    out_sd = jax.ShapeDtypeStruct((E, D), jnp.float32)

    @functools.partial(
        pl.kernel,
        out_type=(out_sd,) * 4,
        mesh=_MESH,
        scratch_types=[
            pltpu.VMEM((C,), jnp.int32),
            pltpu.VMEM((C, D), jnp.float32),
            pltpu.SemaphoreType.DMA,
        ],
    )
    def k(tab_h, i0_h, i1_h, i2_h, i3_h, o0_h, o1_h, o2_h, o3_h,
          idx_v, rows_v, sem):
        base = _worker_id() * per_w
        for idx_h, out_h in ((i0_h, o0_h), (i1_h, o1_h),
                             (i2_h, o2_h), (i3_h, o3_h)):
            def body(i, _, idx_h=idx_h, out_h=out_h):
                off = base + i * C
                pltpu.sync_copy(idx_h.at[pl.ds(off, C)], idx_v)
                pltpu.async_copy(tab_h.at[idx_v], rows_v, sem).wait()
                pltpu.sync_copy(rows_v, out_h.at[pl.ds(off, C)])
                return 0
            lax.fori_loop(0, n_chunks, body, 0)

    return k(table, i0, i1, i2, i3)


NSEG = N_ENT * N_REL
SEG_W = NSEG // NS
CNT_C = 80
STAGE_C = 2000


def _seg_counts(seg, zeros_seg):
    per_w = E // NW
    n_chunks = per_w // CNT_C

    @functools.partial(
        pl.kernel,
        out_type=(jax.ShapeDtypeStruct((NSEG,), jnp.float32),) * 2,
        mesh=_MESH,
        scratch_types=[
            pltpu.VMEM((CNT_C,), jnp.int32),
            pltpu.VMEM((CNT_C,), jnp.float32),
            pltpu.VMEM((STAGE_C,), jnp.float32),
            pltpu.VMEM_SHARED((NSEG,), jnp.float32),
        ],
    )
    def k(seg_h, zero_h, cnt0_h, cnt1_h, seg_v, ones_v, stage_v, cnt_sh):
        cid = lax.axis_index("c")
        sid = lax.axis_index("s")
        base = _worker_id() * per_w

        for j in range(CNT_C // 16):
            ones_v[pl.ds(j * 16, 16)] = jnp.ones((16,), jnp.float32)

        def zstage(i, _):
            stage_v[pl.ds(i * 16, 16)] = jnp.zeros((16,), jnp.float32)
            return 0
        lax.fori_loop(0, STAGE_C // 16, zstage, 0)
        for j in range(SEG_W // STAGE_C):
            pltpu.sync_copy(
                stage_v, cnt_sh.at[pl.ds(sid * SEG_W + j * STAGE_C, STAGE_C)])
        plsc.subcore_barrier()

        def body(i, _):
            pltpu.sync_copy(seg_h.at[pl.ds(base + i * CNT_C, CNT_C)], seg_v)
            pltpu.sync_copy(ones_v, cnt_sh.at[seg_v], add=True)
            return 0
        lax.fori_loop(0, n_chunks, body, 0)

        plsc.subcore_barrier()
        for j in range(SEG_W // STAGE_C):
            off = sid * SEG_W + j * STAGE_C
            pltpu.sync_copy(cnt_sh.at[pl.ds(off, STAGE_C)], stage_v)

            @pl.when(cid == 0)
            def _(off=off):
                pltpu.sync_copy(stage_v, cnt0_h.at[pl.ds(off, STAGE_C)])

            @pl.when(cid == 1)
            def _(off=off):
                pltpu.sync_copy(stage_v, cnt1_h.at[pl.ds(off, STAGE_C)])

    return k(seg, zeros_seg)


AGG_C = 80
PK = 3 * AGG_C
DUMP_C = 200
DUMP_N = N_ENT // DUMP_C


def _conv_agg(t_flat, idx_pk, inv, zeros_nd):
    per_w = E // NW
    n_chunks = per_w // AGG_C
    n_pairs = (n_chunks - 1) // 2

    @functools.partial(
        pl.kernel,
        out_type=(jax.ShapeDtypeStruct((N_ENT, D), jnp.float32),) * 2,
        mesh=_MESH,
        scratch_types=[
            pltpu.VMEM((PK,), jnp.int32),
            pltpu.VMEM((PK,), jnp.int32),
            pltpu.VMEM((AGG_C,), jnp.int32),
            pltpu.VMEM((AGG_C,), jnp.int32),
            pltpu.VMEM((AGG_C,), jnp.float32),
            pltpu.VMEM((AGG_C,), jnp.float32),
            pltpu.VMEM((AGG_C, D), jnp.float32),
            pltpu.VMEM((AGG_C, D), jnp.float32),
            pltpu.VMEM((DUMP_C, D), jnp.float32),
            pltpu.VMEM_SHARED((N_ENT, D), jnp.float32),
        ] + [pltpu.SemaphoreType.DMA] * 8,
    )
    def k(t_h, idx_h, inv_h, zero_h, out0_h, out1_h,
          idx0, idx1, dstc0, dstc1, scl0, scl1, rows0, rows1, stage_v,
          acc_sh, s_i0, s_i1, s_r0, s_r1, s_c0, s_c1, s_o0, s_o1):
        cid = lax.axis_index("c")
        sid = lax.axis_index("s")
        base_chunk = _worker_id() * n_chunks

        idx_bufs = (idx0, idx1)
        dst_bufs = (dstc0, dstc1)
        scl_bufs = (scl0, scl1)
        row_bufs = (rows0, rows1)
        i_sems = (s_i0, s_i1)
        r_sems = (s_r0, s_r1)
        c_sems = (s_c0, s_c1)
        o_sems = (s_o0, s_o1)

        def coff(i):
            c = base_chunk + jnp.minimum(i, n_chunks - 1)
            return pl.multiple_of(c * PK, 8)

        def fire_idx(i, b):
            pltpu.async_copy(idx_h.at[pl.ds(coff(i), PK)],
                             idx_bufs[b], i_sems[b])

        def wait_idx(b):
            pltpu.make_async_copy(idx_h.at[pl.ds(0, PK)],
                                  idx_bufs[b], i_sems[b]).wait()

        def fire_fetch(b):
            ib = idx_bufs[b]
            pltpu.async_copy(t_h.at[ib.at[pl.ds(0, AGG_C)]],
                             row_bufs[b], r_sems[b])
            pltpu.async_copy(inv_h.at[ib.at[pl.ds(AGG_C, AGG_C)]],
                             scl_bufs[b], c_sems[b])

        def wait_fetch(b):
            pltpu.make_async_copy(t_h.at[idx_bufs[b].at[pl.ds(0, AGG_C)]],
                                  row_bufs[b], r_sems[b]).wait()
            pltpu.make_async_copy(
                inv_h.at[idx_bufs[b].at[pl.ds(AGG_C, AGG_C)]],
                scl_bufs[b], c_sems[b]).wait()

        def copy_dst(b):
            for q in range(AGG_C // 16):
                dst_bufs[b][pl.ds(q * 16, 16)] = (
                    idx_bufs[b][pl.ds(2 * AGG_C + q * 16, 16)])

        def scale_rows(b):
            rows_v, scale_v = row_bufs[b], scl_bufs[b]

            def grp(g, _):
                sv = scale_v[pl.ds(g * 16, 16)]
                for l in range(16):
                    s = sv.at[jnp.full((16,), l, jnp.int32)].get(
                        mode="promise_in_bounds")
                    e = g * 16 + l
                    for j in range(D // 16):
                        rows_v[e, pl.ds(j * 16, 16)] = (
                            rows_v[e, pl.ds(j * 16, 16)] * s)
                return 0
            lax.fori_loop(0, AGG_C // 16, grp, 0)

        def fire_scatter(b):
            pltpu.async_copy(row_bufs[b], acc_sh.at[dst_bufs[b]],
                             o_sems[b], add=True)

        def wait_scatter(b):
            pltpu.make_async_copy(row_bufs[b], acc_sh.at[dst_bufs[b]],
                                  o_sems[b]).wait()

        pltpu.sync_copy(zero_h.at[pl.ds(0, DUMP_C)], stage_v)
        for j in range(DUMP_N // NS + 1):
            c = sid + j * NS

            @pl.when(c < DUMP_N)
            def _(c=c):
                off = pl.multiple_of(c * DUMP_C, 8)
                pltpu.sync_copy(stage_v, acc_sh.at[pl.ds(off, DUMP_C)])
        plsc.subcore_barrier()

        fire_idx(0, 0)
        fire_idx(1, 1)
        wait_idx(0)
        fire_fetch(0)

        def pair(j, _):
            wait_fetch(0)
            wait_idx(1)

            @pl.when(j > 0)
            def _():
                wait_scatter(1)
            fire_fetch(1)
            copy_dst(0)
            fire_idx(2 * j + 2, 0)
            scale_rows(0)
            fire_scatter(0)
            wait_fetch(1)
            wait_idx(0)
            wait_scatter(0)
            fire_fetch(0)
            copy_dst(1)
            fire_idx(2 * j + 3, 1)
            scale_rows(1)
            fire_scatter(1)
            return 0
        lax.fori_loop(0, n_pairs, pair, 0)

        wait_fetch(0)
        copy_dst(0)
        scale_rows(0)
        fire_scatter(0)

        wait_scatter(0)
        wait_scatter(1)
        wait_idx(1)

        plsc.subcore_barrier()
        for j in range(DUMP_N // NS + 1):
            c = sid + j * NS

            @pl.when(c < DUMP_N)
            def _(c=c):
                off = pl.multiple_of(c * DUMP_C, 8)
                pltpu.sync_copy(acc_sh.at[pl.ds(off, DUMP_C)], stage_v)

                @pl.when(cid == 0)
                def _():
                    pltpu.sync_copy(stage_v, out0_h.at[pl.ds(off, DUMP_C)])

                @pl.when(cid == 1)
                def _():
                    pltpu.sync_copy(stage_v, out1_h.at[pl.ds(off, DUMP_C)])

    return k(t_flat, idx_pk, inv, zeros_nd)


def _prep(x, wstack):
    def body(x_ref, w_ref, o_ref):
        o_ref[0] = jnp.dot(x_ref[...], w_ref[0],
                           preferred_element_type=jnp.float32)

    return pl.pallas_call(
        body,
        grid=(N_REL + 1,),
        in_specs=[
            pl.BlockSpec((N_ENT, D), lambda r: (0, 0)),
            pl.BlockSpec((1, D, D), lambda r: (r, 0, 0)),
        ],
        out_specs=pl.BlockSpec((1, N_ENT, D), lambda r: (r, 0, 0)),
        out_shape=jax.ShapeDtypeStruct((N_REL + 1, N_ENT, D), jnp.float32),
    )(x, wstack)


def _inv_counts(c0, c1):
    def body(a_ref, b_ref, o_ref):
        o_ref[...] = 1.0 / jnp.maximum(a_ref[...] + b_ref[...], 1.0)

    r = NSEG // D
    out = pl.pallas_call(
        body,
        out_shape=jax.ShapeDtypeStruct((r, D), jnp.float32),
    )(c0.reshape(r, D), c1.reshape(r, D))
    return out.reshape(NSEG)


def _combine(p0, p1, xr, bias, relu, with_ssq):
    def body(a_ref, b_ref, c_ref, bias_ref, o_ref, *maybe_ssq):
        v = a_ref[...] + b_ref[...] + c_ref[...] + bias_ref[...]
        if relu:
            v = jnp.maximum(v, 0.0)
        o_ref[...] = v
        if with_ssq:
            maybe_ssq[0][...] = jnp.sum(v * v).reshape(1, 1)

    out_shape = [jax.ShapeDtypeStruct((N_ENT, D), jnp.float32)]
    if with_ssq:
        out_shape.append(jax.ShapeDtypeStruct((1, 1), jnp.float32))
    res = pl.pallas_call(
        body,
        out_shape=tuple(out_shape),
    )(p0, p1, xr, bias.reshape(1, D))
    return res


LOSS_CH = 4000
LOSS_N = E // LOSS_CH


def _loss(zs_p, zd_p, zs_n, zd_n, edge_type, rel_emb):
    def body(sp_ref, dp_ref, sn_ref, dn_ref, et_ref, rel_ref, o_ref, r2_ref):
        i = pl.program_id(0)
        et = et_ref[0, 0, :]
        onehot = (et[:, None] ==
                  lax.broadcasted_iota(jnp.int32, (LOSS_CH, N_REL), 1)
                  ).astype(jnp.float32)
        rele = jnp.dot(onehot, rel_ref[...],
                       preferred_element_type=jnp.float32)
        s_pos = jnp.sum(sp_ref[...] * rele * dp_ref[...], axis=1)
        s_neg = jnp.sum(sn_ref[...] * rele * dn_ref[...], axis=1)
        part = (jnp.sum(jax.nn.softplus(-s_pos))
                + jnp.sum(jax.nn.softplus(s_neg)))

        @pl.when(i == 0)
        def _():
            o_ref[...] = jnp.zeros((1, 1), jnp.float32)
            r2_ref[...] = jnp.sum(rel_ref[...] * rel_ref[...]).reshape(1, 1)

        o_ref[...] += part.reshape(1, 1)

    row = pl.BlockSpec((LOSS_CH, D), lambda i: (i, 0))
    tot, relsq = pl.pallas_call(
        body,
        grid=(LOSS_N,),
        in_specs=[
            row, row, row, row,
            pl.BlockSpec((1, 1, LOSS_CH), lambda i: (i, 0, 0)),
            pl.BlockSpec((N_REL, D), lambda i: (0, 0)),
        ],
        out_specs=(pl.BlockSpec((1, 1), lambda i: (0, 0)),
                   pl.BlockSpec((1, 1), lambda i: (0, 0))),
        out_shape=(jax.ShapeDtypeStruct((1, 1), jnp.float32),
                   jax.ShapeDtypeStruct((1, 1), jnp.float32)),
    )(zs_p, zd_p, zs_n, zd_n, edge_type.reshape(LOSS_N, 1, LOSS_CH), rel_emb)
    return tot[0, 0], relsq[0, 0]


def _block_diag_stack(weight, root):
    bs = D // N_BLOCKS
    bd = jnp.zeros((N_REL, D, D), jnp.float32)
    for b in range(N_BLOCKS):
        bd = bd.at[:, b * bs:(b + 1) * bs, b * bs:(b + 1) * bs].set(
            weight[:, b])
    return jnp.concatenate([bd, root[None]], axis=0)


def _rgcn_conv_sc(x, idx_pk, inv, zeros_nd, weight, root, bias,
                  relu, with_ssq):
    s = _prep(x, _block_diag_stack(weight, root))
    t_flat = s[:N_REL].reshape(N_REL * N_ENT, D)
    p0, p1 = _conv_agg(t_flat, idx_pk, inv, zeros_nd)
    return _combine(p0, p1, s[N_REL], bias, relu, with_ssq)


def kernel(edge_pos, edge_neg, edge_type, node_emb, conv1_weight, conv1_root,
           conv1_bias, conv2_weight, conv2_root, conv2_bias, rel_emb):
    src = edge_pos[0]
    dst = edge_pos[1]
    seg = dst * N_REL + edge_type
    gidx = edge_type * N_ENT + src
    idx_pk = jnp.concatenate(
        [gidx.reshape(-1, 1, AGG_C), seg.reshape(-1, 1, AGG_C),
         dst.reshape(-1, 1, AGG_C)], axis=1).reshape(-1)
    zeros_nd = jnp.zeros((N_ENT, D), jnp.float32)

    c0, c1 = _seg_counts(seg, jnp.zeros((NSEG,), jnp.float32))
    inv = _inv_counts(c0, c1)

    (x,) = _rgcn_conv_sc(node_emb, idx_pk, inv, zeros_nd,
                         conv1_weight, conv1_root, conv1_bias,
                         relu=True, with_ssq=False)
    z, ssq = _rgcn_conv_sc(x, idx_pk, inv, zeros_nd,
                           conv2_weight, conv2_root, conv2_bias,
                           relu=False, with_ssq=True)

    zs_p, zd_p, zs_n, zd_n = _gather4(
        z, edge_pos[0], edge_pos[1], edge_neg[0], edge_neg[1])
    tot, relsq = _loss(zs_p, zd_p, zs_n, zd_n, edge_type, rel_emb)

    ce = tot / (2.0 * E)
    reg = ssq[0, 0] / (N_ENT * D) + relsq / (N_REL * D)
    return ce + 0.01 * reg


# speedup vs baseline: 6.1425x; 1.4366x over previous
import functools

import jax
import jax.numpy as jnp
from jax import lax
from jax.experimental import pallas as pl
from jax.experimental.pallas import tpu as pltpu
from jax.experimental.pallas import tpu_sc as plsc

N_ENT = 10000
N_REL = 16
D = 128
N_BLOCKS = 4
E = 320000

NC = 2
NS = 16
NW = NC * NS

_MESH = plsc.VectorSubcoreMesh(core_axis_name="c", subcore_axis_name="s")


def _worker_id():
    return lax.axis_index("s") * NC + lax.axis_index("c")


def _gather_all(table, idx4):
    C = 80
    B4 = 4 * E
    per_w = B4 // NW
    n_chunks = per_w // C
    n_quads = n_chunks // 4

    @functools.partial(
        pl.kernel,
        out_type=jax.ShapeDtypeStruct((B4, D), jnp.float32),
        mesh=_MESH,
        scratch_types=(
            [pltpu.VMEM((C,), jnp.int32)] * 4
            + [pltpu.VMEM((C, D), jnp.float32)] * 4
            + [pltpu.SemaphoreType.DMA] * 12
        ),
    )
    def k(tab_h, idx_h, out_h, i0, i1, i2, i3, r0, r1, r2, r3, *sems):
        s_i = sems[0:4]
        s_g = sems[4:8]
        s_w = sems[8:12]
        idx_b = (i0, i1, i2, i3)
        row_b = (r0, r1, r2, r3)
        base = _worker_id() * per_w

        def off(c):
            return pl.multiple_of(base + c * C, 8)

        def fire_idx(c, b):
            pltpu.async_copy(idx_h.at[pl.ds(off(c), C)], idx_b[b], s_i[b])

        def wait_idx(b):
            pltpu.make_async_copy(idx_h.at[pl.ds(0, C)],
                                  idx_b[b], s_i[b]).wait()

        def fire_gather(b):
            pltpu.async_copy(tab_h.at[idx_b[b]], row_b[b], s_g[b])

        def wait_gather(b):
            pltpu.make_async_copy(tab_h.at[idx_b[b]],
                                  row_b[b], s_g[b]).wait()

        def fire_wb(c, b):
            pltpu.async_copy(row_b[b], out_h.at[pl.ds(off(c), C)], s_w[b])

        def wait_wb(b):
            pltpu.make_async_copy(row_b[b], out_h.at[pl.ds(0, C)],
                                  s_w[b]).wait()

        for b in range(4):
            fire_idx(b, b)
        for b in range(3):
            wait_idx(b)
            fire_gather(b)

        def quad(t, _):
            for b in range(4):
                i = 4 * t + b
                wait_gather(b)
                fire_wb(i, b)
                bn = (b + 3) % 4
                if b == 0:
                    @pl.when(t < n_quads - 1)
                    def _():
                        fire_idx(i + 4, b)

                    wait_idx(bn)

                    @pl.when(t > 0)
                    def _():
                        wait_wb(bn)
                    fire_gather(bn)
                else:
                    @pl.when(t < n_quads - 1)
                    def _():
                        fire_idx(i + 4, b)
                        wait_idx(bn)
                        wait_wb(bn)
                        fire_gather(bn)
            return 0
        lax.fori_loop(0, n_quads, quad, 0)

        for b in range(4):
            wait_wb(b)

    return k(table, idx4)


NSEG = N_ENT * N_REL
SEG_W = NSEG // NS
CNT_C = 80
STAGE_C = 2000


def _seg_counts(seg, zeros_seg):
    per_w = E // NW
    n_chunks = per_w // CNT_C

    @functools.partial(
        pl.kernel,
        out_type=(jax.ShapeDtypeStruct((NSEG,), jnp.float32),) * 2,
        mesh=_MESH,
        scratch_types=[
            pltpu.VMEM((CNT_C,), jnp.int32),
            pltpu.VMEM((CNT_C,), jnp.float32),
            pltpu.VMEM((STAGE_C,), jnp.float32),
            pltpu.VMEM_SHARED((NSEG,), jnp.float32),
        ],
    )
    def k(seg_h, zero_h, cnt0_h, cnt1_h, seg_v, ones_v, stage_v, cnt_sh):
        cid = lax.axis_index("c")
        sid = lax.axis_index("s")
        base = _worker_id() * per_w

        for j in range(CNT_C // 16):
            ones_v[pl.ds(j * 16, 16)] = jnp.ones((16,), jnp.float32)

        def zstage(i, _):
            stage_v[pl.ds(i * 16, 16)] = jnp.zeros((16,), jnp.float32)
            return 0
        lax.fori_loop(0, STAGE_C // 16, zstage, 0)
        for j in range(SEG_W // STAGE_C):
            pltpu.sync_copy(
                stage_v, cnt_sh.at[pl.ds(sid * SEG_W + j * STAGE_C, STAGE_C)])
        plsc.subcore_barrier()

        def body(i, _):
            pltpu.sync_copy(seg_h.at[pl.ds(base + i * CNT_C, CNT_C)], seg_v)
            pltpu.sync_copy(ones_v, cnt_sh.at[seg_v], add=True)
            return 0
        lax.fori_loop(0, n_chunks, body, 0)

        plsc.subcore_barrier()
        for j in range(SEG_W // STAGE_C):
            off = sid * SEG_W + j * STAGE_C
            pltpu.sync_copy(cnt_sh.at[pl.ds(off, STAGE_C)], stage_v)

            @pl.when(cid == 0)
            def _(off=off):
                pltpu.sync_copy(stage_v, cnt0_h.at[pl.ds(off, STAGE_C)])

            @pl.when(cid == 1)
            def _(off=off):
                pltpu.sync_copy(stage_v, cnt1_h.at[pl.ds(off, STAGE_C)])

    return k(seg, zeros_seg)


AGG_C = 80
PK = 3 * AGG_C
DUMP_C = 200
DUMP_N = N_ENT // DUMP_C


def _conv_agg(t_flat, idx_pk, inv, zeros_nd):
    per_w = E // NW
    n_chunks = per_w // AGG_C
    n_pairs = (n_chunks - 1) // 2

    @functools.partial(
        pl.kernel,
        out_type=(jax.ShapeDtypeStruct((N_ENT, D), jnp.float32),) * 2,
        mesh=_MESH,
        scratch_types=[
            pltpu.VMEM((PK,), jnp.int32),
            pltpu.VMEM((PK,), jnp.int32),
            pltpu.VMEM((AGG_C,), jnp.int32),
            pltpu.VMEM((AGG_C,), jnp.int32),
            pltpu.VMEM((AGG_C,), jnp.float32),
            pltpu.VMEM((AGG_C,), jnp.float32),
            pltpu.VMEM((AGG_C, D), jnp.float32),
            pltpu.VMEM((AGG_C, D), jnp.float32),
            pltpu.VMEM((DUMP_C, D), jnp.float32),
            pltpu.VMEM_SHARED((N_ENT, D), jnp.float32),
        ] + [pltpu.SemaphoreType.DMA] * 8,
    )
    def k(t_h, idx_h, inv_h, zero_h, out0_h, out1_h,
          idx0, idx1, dstc0, dstc1, scl0, scl1, rows0, rows1, stage_v,
          acc_sh, s_i0, s_i1, s_r0, s_r1, s_c0, s_c1, s_o0, s_o1):
        cid = lax.axis_index("c")
        sid = lax.axis_index("s")
        base_chunk = _worker_id() * n_chunks

        idx_bufs = (idx0, idx1)
        dst_bufs = (dstc0, dstc1)
        scl_bufs = (scl0, scl1)
        row_bufs = (rows0, rows1)
        i_sems = (s_i0, s_i1)
        r_sems = (s_r0, s_r1)
        c_sems = (s_c0, s_c1)
        o_sems = (s_o0, s_o1)

        def coff(i):
            c = base_chunk + jnp.minimum(i, n_chunks - 1)
            return pl.multiple_of(c * PK, 8)

        def fire_idx(i, b):
            pltpu.async_copy(idx_h.at[pl.ds(coff(i), PK)],
                             idx_bufs[b], i_sems[b])

        def wait_idx(b):
            pltpu.make_async_copy(idx_h.at[pl.ds(0, PK)],
                                  idx_bufs[b], i_sems[b]).wait()

        def fire_fetch(b):
            ib = idx_bufs[b]
            pltpu.async_copy(t_h.at[ib.at[pl.ds(0, AGG_C)]],
                             row_bufs[b], r_sems[b])
            pltpu.async_copy(inv_h.at[ib.at[pl.ds(AGG_C, AGG_C)]],
                             scl_bufs[b], c_sems[b])

        def wait_fetch(b):
            pltpu.make_async_copy(t_h.at[idx_bufs[b].at[pl.ds(0, AGG_C)]],
                                  row_bufs[b], r_sems[b]).wait()
            pltpu.make_async_copy(
                inv_h.at[idx_bufs[b].at[pl.ds(AGG_C, AGG_C)]],
                scl_bufs[b], c_sems[b]).wait()

        def copy_dst(b):
            for q in range(AGG_C // 16):
                dst_bufs[b][pl.ds(q * 16, 16)] = (
                    idx_bufs[b][pl.ds(2 * AGG_C + q * 16, 16)])

        def scale_rows(b):
            rows_v, scale_v = row_bufs[b], scl_bufs[b]

            def grp(g, _):
                sv = scale_v[pl.ds(g * 16, 16)]
                for l in range(16):
                    s = sv.at[jnp.full((16,), l, jnp.int32)].get(
                        mode="promise_in_bounds")
                    e = g * 16 + l
                    for j in range(D // 16):
                        rows_v[e, pl.ds(j * 16, 16)] = (
                            rows_v[e, pl.ds(j * 16, 16)] * s)
                return 0
            lax.fori_loop(0, AGG_C // 16, grp, 0)

        def fire_scatter(b):
            pltpu.async_copy(row_bufs[b], acc_sh.at[dst_bufs[b]],
                             o_sems[b], add=True)

        def wait_scatter(b):
            pltpu.make_async_copy(row_bufs[b], acc_sh.at[dst_bufs[b]],
                                  o_sems[b]).wait()

        pltpu.sync_copy(zero_h.at[pl.ds(0, DUMP_C)], stage_v)
        for j in range(DUMP_N // NS + 1):
            c = sid + j * NS

            @pl.when(c < DUMP_N)
            def _(c=c):
                off = pl.multiple_of(c * DUMP_C, 8)
                pltpu.sync_copy(stage_v, acc_sh.at[pl.ds(off, DUMP_C)])
        plsc.subcore_barrier()

        fire_idx(0, 0)
        fire_idx(1, 1)
        wait_idx(0)
        fire_fetch(0)

        def pair(j, _):
            wait_fetch(0)
            wait_idx(1)

            @pl.when(j > 0)
            def _():
                wait_scatter(1)
            fire_fetch(1)
            copy_dst(0)
            fire_idx(2 * j + 2, 0)
            scale_rows(0)
            fire_scatter(0)
            wait_fetch(1)
            wait_idx(0)
            wait_scatter(0)
            fire_fetch(0)
            copy_dst(1)
            fire_idx(2 * j + 3, 1)
            scale_rows(1)
            fire_scatter(1)
            return 0
        lax.fori_loop(0, n_pairs, pair, 0)

        wait_fetch(0)
        copy_dst(0)
        scale_rows(0)
        fire_scatter(0)

        wait_scatter(0)
        wait_scatter(1)
        wait_idx(1)

        plsc.subcore_barrier()
        for j in range(DUMP_N // NS + 1):
            c = sid + j * NS

            @pl.when(c < DUMP_N)
            def _(c=c):
                off = pl.multiple_of(c * DUMP_C, 8)
                pltpu.sync_copy(acc_sh.at[pl.ds(off, DUMP_C)], stage_v)

                @pl.when(cid == 0)
                def _():
                    pltpu.sync_copy(stage_v, out0_h.at[pl.ds(off, DUMP_C)])

                @pl.when(cid == 1)
                def _():
                    pltpu.sync_copy(stage_v, out1_h.at[pl.ds(off, DUMP_C)])

    return k(t_flat, idx_pk, inv, zeros_nd)


def _prep(x, wstack):
    def body(x_ref, w_ref, o_ref):
        o_ref[0] = jnp.dot(x_ref[...], w_ref[0],
                           preferred_element_type=jnp.float32)

    return pl.pallas_call(
        body,
        grid=(N_REL + 1,),
        in_specs=[
            pl.BlockSpec((N_ENT, D), lambda r: (0, 0)),
            pl.BlockSpec((1, D, D), lambda r: (r, 0, 0)),
        ],
        out_specs=pl.BlockSpec((1, N_ENT, D), lambda r: (r, 0, 0)),
        out_shape=jax.ShapeDtypeStruct((N_REL + 1, N_ENT, D), jnp.float32),
    )(x, wstack)


def _inv_counts(c0, c1):
    def body(a_ref, b_ref, o_ref):
        o_ref[...] = 1.0 / jnp.maximum(a_ref[...] + b_ref[...], 1.0)

    r = NSEG // D
    out = pl.pallas_call(
        body,
        out_shape=jax.ShapeDtypeStruct((r, D), jnp.float32),
    )(c0.reshape(r, D), c1.reshape(r, D))
    return out.reshape(NSEG)


def _combine(p0, p1, xr, bias, relu, with_ssq):
    def body(a_ref, b_ref, c_ref, bias_ref, o_ref, *maybe_ssq):
        v = a_ref[...] + b_ref[...] + c_ref[...] + bias_ref[...]
        if relu:
            v = jnp.maximum(v, 0.0)
        o_ref[...] = v
        if with_ssq:
            maybe_ssq[0][...] = jnp.sum(v * v).reshape(1, 1)

    out_shape = [jax.ShapeDtypeStruct((N_ENT, D), jnp.float32)]
    if with_ssq:
        out_shape.append(jax.ShapeDtypeStruct((1, 1), jnp.float32))
    res = pl.pallas_call(
        body,
        out_shape=tuple(out_shape),
    )(p0, p1, xr, bias.reshape(1, D))
    return res


LOSS_CH = 4000
LOSS_N = E // LOSS_CH


def _loss(gall, edge_type, rel_emb):
    def body(sp_ref, dp_ref, sn_ref, dn_ref, et_ref, rel_ref, o_ref, r2_ref):
        i = pl.program_id(0)
        et = et_ref[0, 0, :]
        onehot = (et[:, None] ==
                  lax.broadcasted_iota(jnp.int32, (LOSS_CH, N_REL), 1)
                  ).astype(jnp.float32)
        rele = jnp.dot(onehot, rel_ref[...],
                       preferred_element_type=jnp.float32)
        s_pos = jnp.sum(sp_ref[...] * rele * dp_ref[...], axis=1)
        s_neg = jnp.sum(sn_ref[...] * rele * dn_ref[...], axis=1)
        part = (jnp.sum(jax.nn.softplus(-s_pos))
                + jnp.sum(jax.nn.softplus(s_neg)))

        @pl.when(i == 0)
        def _():
            o_ref[...] = jnp.zeros((1, 1), jnp.float32)
            r2_ref[...] = jnp.sum(rel_ref[...] * rel_ref[...]).reshape(1, 1)

        o_ref[...] += part.reshape(1, 1)

    nb = E // LOSS_CH
    tot, relsq = pl.pallas_call(
        body,
        grid=(LOSS_N,),
        in_specs=[
            pl.BlockSpec((LOSS_CH, D), lambda i: (i, 0)),
            pl.BlockSpec((LOSS_CH, D), lambda i: (nb + i, 0)),
            pl.BlockSpec((LOSS_CH, D), lambda i: (2 * nb + i, 0)),
            pl.BlockSpec((LOSS_CH, D), lambda i: (3 * nb + i, 0)),
            pl.BlockSpec((1, 1, LOSS_CH), lambda i: (i, 0, 0)),
            pl.BlockSpec((N_REL, D), lambda i: (0, 0)),
        ],
        out_specs=(pl.BlockSpec((1, 1), lambda i: (0, 0)),
                   pl.BlockSpec((1, 1), lambda i: (0, 0))),
        out_shape=(jax.ShapeDtypeStruct((1, 1), jnp.float32),
                   jax.ShapeDtypeStruct((1, 1), jnp.float32)),
    )(gall, gall, gall, gall,
      edge_type.reshape(LOSS_N, 1, LOSS_CH), rel_emb)
    return tot[0, 0], relsq[0, 0]


def _block_diag_stack(weight, root):
    bs = D // N_BLOCKS
    bd = jnp.zeros((N_REL, D, D), jnp.float32)
    for b in range(N_BLOCKS):
        bd = bd.at[:, b * bs:(b + 1) * bs, b * bs:(b + 1) * bs].set(
            weight[:, b])
    return jnp.concatenate([bd, root[None]], axis=0)


def _rgcn_conv_sc(x, idx_pk, inv, zeros_nd, weight, root, bias,
                  relu, with_ssq):
    s = _prep(x, _block_diag_stack(weight, root))
    t_flat = s[:N_REL].reshape(N_REL * N_ENT, D)
    p0, p1 = _conv_agg(t_flat, idx_pk, inv, zeros_nd)
    return _combine(p0, p1, s[N_REL], bias, relu, with_ssq)


def kernel(edge_pos, edge_neg, edge_type, node_emb, conv1_weight, conv1_root,
           conv1_bias, conv2_weight, conv2_root, conv2_bias, rel_emb):
    src = edge_pos[0]
    dst = edge_pos[1]
    seg = dst * N_REL + edge_type
    gidx = edge_type * N_ENT + src
    idx_pk = jnp.concatenate(
        [gidx.reshape(-1, 1, AGG_C), seg.reshape(-1, 1, AGG_C),
         dst.reshape(-1, 1, AGG_C)], axis=1).reshape(-1)
    zeros_nd = jnp.zeros((N_ENT, D), jnp.float32)

    c0, c1 = _seg_counts(seg, jnp.zeros((NSEG,), jnp.float32))
    inv = _inv_counts(c0, c1)

    (x,) = _rgcn_conv_sc(node_emb, idx_pk, inv, zeros_nd,
                         conv1_weight, conv1_root, conv1_bias,
                         relu=True, with_ssq=False)
    z, ssq = _rgcn_conv_sc(x, idx_pk, inv, zeros_nd,
                           conv2_weight, conv2_root, conv2_bias,
                           relu=False, with_ssq=True)

    idx4 = jnp.concatenate([edge_pos[0], edge_pos[1],
                            edge_neg[0], edge_neg[1]])
    gall = _gather_all(z, idx4)
    tot, relsq = _loss(gall, edge_type, rel_emb)

    ce = tot / (2.0 * E)
    reg = ssq[0, 0] / (N_ENT * D) + relsq / (N_REL * D)
    return ce + 0.01 * reg
